# Initial kernel scaffold; baseline (speedup 1.0000x reference)
import jax
import jax.numpy as jnp
from jax.experimental import pallas as pl


def kernel(pos_indicies, W):
    raise NotImplementedError("write your pallas kernel here")


# speedup vs baseline: 3.5521x; 3.5521x over previous
import jax
import jax.numpy as jnp
from jax import lax
from jax.experimental import pallas as pl
from jax.experimental.pallas import tpu as pltpu
from jax.experimental.pallas import tpu_sc as plsc

N_TIMESTEPS = 100000
D = 64
B_ROWS = 4096
B_COLS = 200
B_TOT = B_ROWS * B_COLS

NC, NS = 2, 16
NW = NC * NS
PER_W = B_TOT // NW
CHUNK = 128
N_CHUNKS = PER_W // CHUNK


def _gather_body(idx_hbm, table_hbm, out_hbm, idx_v, rows_v, sem):
    wid = lax.axis_index("s") * NC + lax.axis_index("c")
    pltpu.sync_copy(idx_hbm.at[wid], idx_v)

    @pl.loop(0, N_CHUNKS)
    def _chunk(j):
        pltpu.async_copy(table_hbm.at[idx_v.at[j]], rows_v, sem).wait()
        pltpu.sync_copy(rows_v, out_hbm.at[wid, j])


def kernel(pos_indicies, W):
    idx = pos_indicies.astype(jnp.int32).reshape(NW, N_CHUNKS, CHUNK)
    mesh = plsc.VectorSubcoreMesh(core_axis_name="c", subcore_axis_name="s")
    out = pl.kernel(
        _gather_body,
        out_type=jax.ShapeDtypeStruct((NW, N_CHUNKS, CHUNK, D), jnp.float32),
        mesh=mesh,
        scratch_types=[
            pltpu.VMEM((N_CHUNKS, CHUNK), jnp.int32),
            pltpu.VMEM((CHUNK, D), jnp.float32),
            pltpu.SemaphoreType.DMA,
        ],
        compiler_params=pltpu.CompilerParams(use_tc_tiling_on_sc=False),
    )(idx, W)
    return out.reshape(B_ROWS, B_COLS, D)


# speedup vs baseline: 4.2672x; 1.2013x over previous
import jax
import jax.numpy as jnp
from jax import lax
from jax.experimental import pallas as pl
from jax.experimental.pallas import tpu as pltpu
from jax.experimental.pallas import tpu_sc as plsc

N_TIMESTEPS = 100000
D = 64
B_ROWS = 4096
B_COLS = 200
B_TOT = B_ROWS * B_COLS

NC, NS = 2, 16
NW = NC * NS
PER_W = B_TOT // NW
CHUNK = 128
N_CHUNKS = PER_W // CHUNK
NBUF = 8
PF = 4
N_GROUPS = N_CHUNKS // NBUF


def _gather_body(idx_hbm, table_hbm, out_hbm, idx_v, rows_v, gsem, wsem):
    wid = lax.axis_index("s") * NC + lax.axis_index("c")
    pltpu.sync_copy(idx_hbm.at[wid], idx_v)

    def fire_gather(j, b):
        pltpu.async_copy(table_hbm.at[idx_v.at[j]], rows_v.at[b], gsem.at[b])

    def wait_gather(j, b):
        pltpu.make_async_copy(
            table_hbm.at[idx_v.at[j]], rows_v.at[b], gsem.at[b]).wait()

    def fire_write(j, b):
        pltpu.async_copy(rows_v.at[b], out_hbm.at[wid, j], wsem.at[b])

    def wait_write(j, b):
        pltpu.make_async_copy(
            rows_v.at[b], out_hbm.at[wid, j], wsem.at[b]).wait()

    for b in range(PF):
        fire_gather(b, b)

    @pl.loop(0, N_GROUPS)
    def _group(g):
        base = g * NBUF
        for b in range(NBUF):
            j = base + b
            wait_gather(j, b)
            fire_write(j, b)
            if b < NBUF - PF:
                b2 = b + PF

                @pl.when(g > 0)
                def _():
                    wait_write(base + b2 - NBUF, b2)

                fire_gather(base + b2, b2)
            else:
                b2 = b + PF - NBUF

                @pl.when(g < N_GROUPS - 1)
                def _():
                    wait_write(base + b2, b2)
                    fire_gather(base + NBUF + b2, b2)

    for b in range(NBUF):
        wait_write((N_GROUPS - 1) * NBUF + b, b)


def kernel(pos_indicies, W):
    idx = pos_indicies.astype(jnp.int32).reshape(NW, N_CHUNKS, CHUNK)
    mesh = plsc.VectorSubcoreMesh(core_axis_name="c", subcore_axis_name="s")
    out = pl.kernel(
        _gather_body,
        out_type=jax.ShapeDtypeStruct((NW, N_CHUNKS, CHUNK, D), jnp.float32),
        mesh=mesh,
        scratch_types=[
            pltpu.VMEM((N_CHUNKS, CHUNK), jnp.int32),
            pltpu.VMEM((NBUF, CHUNK, D), jnp.float32),
            pltpu.SemaphoreType.DMA((NBUF,)),
            pltpu.SemaphoreType.DMA((NBUF,)),
        ],
        compiler_params=pltpu.CompilerParams(use_tc_tiling_on_sc=False),
    )(idx, W)
    return out.reshape(B_ROWS, B_COLS, D)
